# baseline (speedup 1.0000x reference)
import functools

import jax
import jax.numpy as jnp
from jax import lax
from jax.experimental import pallas as pl
from jax.experimental.pallas import tpu as pltpu
from jax.experimental.pallas import tpu_sc as plsc

F = 26
V = 100000
D = 16
B = 4096
NC = 2
NS = 16
NW = NC * NS
SPW = B // NW
W = 432
G4 = 4
NG = SPW // G4


def _take16(vec, idx):
    return vec.at[idx].get(mode="promise_in_bounds")


@functools.partial(
    pl.kernel,
    out_type=jax.ShapeDtypeStruct((B,), jnp.float32),
    mesh=plsc.VectorSubcoreMesh(core_axis_name="c", subcore_axis_name="s"),
    compiler_params=pltpu.CompilerParams(use_tc_tiling_on_sc=False),
    scratch_types=[
        pltpu.VMEM((SPW * F,), jnp.int32),
        pltpu.VMEM((G4 * F, W), jnp.float32),
        pltpu.VMEM((G4 * F, W), jnp.float32),
        pltpu.VMEM((SPW,), jnp.float32),
        pltpu.VMEM((16,), jnp.float32),
        pltpu.SemaphoreType.DMA,
        pltpu.SemaphoreType.DMA,
    ],
)
def _ffm_sc(x_hbm, tab_hbm, bias_hbm, out_hbm,
            x_v, slabs0_v, slabs1_v, out_v, bias_v, sem0, sem1):
    wid = lax.axis_index("s") * NC + lax.axis_index("c")
    base = wid * SPW

    pltpu.sync_copy(bias_hbm, bias_v)
    pltpu.sync_copy(x_hbm.at[pl.ds(base * F, SPW * F)], x_v)

    lane = lax.iota(jnp.int32, 16)

    def fire(g, slabs_ref, sem):
        idx = x_v.at[pl.ds(g * (G4 * F), G4 * F)]
        pltpu.async_copy(tab_hbm.at[idx], slabs_ref, sem)

    def wait(g, slabs_ref, sem):
        idx = x_v.at[pl.ds(g * (G4 * F), G4 * F)]
        pltpu.make_async_copy(tab_hbm.at[idx], slabs_ref, sem).wait()

    def compute(g, slabs_ref, out_vec):
        for r in range(G4):
            b = g * G4 + r
            acc = jnp.zeros((16,), jnp.float32)
            for i in range(F - 1):
                for j in range(i + 1, F):
                    acc = acc + (slabs_ref[r * F + i, pl.ds(j * D, D)] *
                                 slabs_ref[r * F + j, pl.ds(i * D, D)])
            accf = slabs_ref[r * F, pl.ds(F * D, D)]
            for s in range(1, F):
                accf = accf + slabs_ref[r * F + s, pl.ds(F * D, D)]
            acc = acc + jnp.where(lane == 0, accf, 0.0)
            for sh in (8, 4, 2, 1):
                acc = acc + _take16(acc, lane ^ sh)
            out_vec = jnp.where(lane == b % 16, acc, out_vec)
            out_v[pl.ds((b // 16) * 16, 16)] = out_vec
        return out_vec

    fire(0, slabs0_v, sem0)

    def pair_body(gg, out_vec):
        g0 = 2 * gg
        g1 = g0 + 1
        fire(g1, slabs1_v, sem1)
        wait(g0, slabs0_v, sem0)
        out_vec = compute(g0, slabs0_v, out_vec)
        fire(jnp.minimum(g1 + 1, NG - 1), slabs0_v, sem0)
        wait(g1, slabs1_v, sem1)
        return compute(g1, slabs1_v, out_vec)

    lax.fori_loop(0, NG // 2, pair_body, jnp.zeros((16,), jnp.float32))
    wait(NG - 1, slabs0_v, sem0)

    bb = bias_v[:]
    for g in range(SPW // 16):
        zz = out_v[pl.ds(g * 16, 16)] + bb
        out_v[pl.ds(g * 16, 16)] = 1.0 / (1.0 + jnp.exp(-zz))
    pltpu.sync_copy(out_v, out_hbm.at[pl.ds(base, SPW)])


BV = 800


def _relayout_body(emb_ref, fc_ref, out_ref):
    blk = emb_ref[...]
    out_ref[:, :F, :] = jnp.swapaxes(blk, 0, 1)
    out_ref[:, F:, :] = jnp.broadcast_to(fc_ref[...][:, :, None],
                                         (BV, W // D - F, D))


_relayout = pl.pallas_call(
    _relayout_body,
    grid=(V // BV,),
    in_specs=[pl.BlockSpec((F, BV, D), lambda i: (0, i, 0)),
              pl.BlockSpec((BV, 1), lambda i: (i, 0))],
    out_specs=pl.BlockSpec((BV, W // D, D), lambda i: (i, 0, 0)),
    out_shape=jax.ShapeDtypeStruct((V, W // D, D), jnp.float32),
)


def kernel(x, emb_tables, fc_table, bias):
    xflat = x.astype(jnp.int32).reshape(B * F)
    tab_aug = _relayout(emb_tables, fc_table.astype(jnp.float32)).reshape(V, W)
    bias16 = jnp.broadcast_to(bias.astype(jnp.float32), (16,))
    return _ffm_sc(xflat, tab_aug, bias16)


# speedup vs baseline: 2.4362x; 2.4362x over previous
import functools

import jax
import jax.numpy as jnp
from jax import lax
from jax.experimental import pallas as pl
from jax.experimental.pallas import tpu as pltpu
from jax.experimental.pallas import tpu_sc as plsc

F = 26
V = 100000
D = 16
B = 4096
NC = 2
NS = 16
NW = NC * NS
SPW = B // NW
W = 432
G4 = 4
NG = SPW // G4


def _take16(vec, idx):
    return vec.at[idx].get(mode="promise_in_bounds")


@functools.partial(
    pl.kernel,
    out_type=jax.ShapeDtypeStruct((B,), jnp.float32),
    mesh=plsc.VectorSubcoreMesh(core_axis_name="c", subcore_axis_name="s"),
    compiler_params=pltpu.CompilerParams(use_tc_tiling_on_sc=False),
    scratch_types=[
        pltpu.VMEM((SPW * F,), jnp.int32),
        pltpu.VMEM((G4 * F, W), jnp.float32),
        pltpu.VMEM((G4 * F, W), jnp.float32),
        pltpu.VMEM((SPW,), jnp.float32),
        pltpu.VMEM((16,), jnp.float32),
        pltpu.SemaphoreType.DMA,
        pltpu.SemaphoreType.DMA,
    ],
)
def _ffm_sc(x_hbm, tab_hbm, bias_hbm, out_hbm,
            x_v, slabs0_v, slabs1_v, out_v, bias_v, sem0, sem1):
    wid = lax.axis_index("s") * NC + lax.axis_index("c")
    base = wid * SPW

    pltpu.sync_copy(bias_hbm, bias_v)
    pltpu.sync_copy(x_hbm.at[pl.ds(base * F, SPW * F)], x_v)

    lane = lax.iota(jnp.int32, 16)

    def fire(g, slabs_ref, sem):
        idx = x_v.at[pl.ds(g * (G4 * F), G4 * F)]
        pltpu.async_copy(tab_hbm.at[idx], slabs_ref, sem)

    def wait(g, slabs_ref, sem):
        idx = x_v.at[pl.ds(g * (G4 * F), G4 * F)]
        pltpu.make_async_copy(tab_hbm.at[idx], slabs_ref, sem).wait()

    def compute(g, slabs_ref, out_vec):
        for r in range(G4):
            b = g * G4 + r
            acc = jnp.zeros((16,), jnp.float32)
            for i in range(F - 1):
                for j in range(i + 1, F):
                    acc = acc + (slabs_ref[r * F + i, pl.ds(j * D, D)] *
                                 slabs_ref[r * F + j, pl.ds(i * D, D)])
            accf = slabs_ref[r * F, pl.ds(F * D, D)]
            for s in range(1, F):
                accf = accf + slabs_ref[r * F + s, pl.ds(F * D, D)]
            acc = acc + jnp.where(lane == 0, accf, 0.0)
            for sh in (8, 4, 2, 1):
                acc = acc + _take16(acc, lane ^ sh)
            out_vec = jnp.where(lane == b % 16, acc, out_vec)
            out_v[pl.ds((b // 16) * 16, 16)] = out_vec
        return out_vec

    fire(0, slabs0_v, sem0)

    def pair_body(gg, out_vec):
        g0 = 2 * gg
        g1 = g0 + 1
        fire(g1, slabs1_v, sem1)
        wait(g0, slabs0_v, sem0)
        out_vec = compute(g0, slabs0_v, out_vec)
        fire(jnp.minimum(g1 + 1, NG - 1), slabs0_v, sem0)
        wait(g1, slabs1_v, sem1)
        return compute(g1, slabs1_v, out_vec)

    lax.fori_loop(0, NG // 2, pair_body, jnp.zeros((16,), jnp.float32))
    wait(NG - 1, slabs0_v, sem0)

    bb = bias_v[:]
    for g in range(SPW // 16):
        zz = out_v[pl.ds(g * 16, 16)] + bb
        out_v[pl.ds(g * 16, 16)] = 1.0 / (1.0 + jnp.exp(-zz))
    pltpu.sync_copy(out_v, out_hbm.at[pl.ds(base, SPW)])


VT = V // NW
CH = 25
NCH = VT // CH


@functools.partial(
    pl.kernel,
    out_type=jax.ShapeDtypeStruct((V * W,), jnp.float32),
    mesh=plsc.VectorSubcoreMesh(core_axis_name="c", subcore_axis_name="s"),
    compiler_params=pltpu.CompilerParams(use_tc_tiling_on_sc=False),
    scratch_types=[
        pltpu.VMEM((F * CH * D,), jnp.float32),
        pltpu.VMEM((F * CH * D,), jnp.float32),
        pltpu.VMEM((32,), jnp.float32),
        pltpu.VMEM((32,), jnp.float32),
        pltpu.VMEM((CH * W,), jnp.float32),
        pltpu.VMEM((CH * W,), jnp.float32),
        pltpu.SemaphoreType.DMA,
        pltpu.SemaphoreType.DMA,
        pltpu.SemaphoreType.DMA,
        pltpu.SemaphoreType.DMA,
    ],
)
def _relayout_sc(emb_hbm, fc_hbm, tab_hbm,
                 in0_v, in1_v, f0_v, f1_v, o0_v, o1_v, si0, si1, so0, so1):
    wid = lax.axis_index("s") * NC + lax.axis_index("c")
    vbase = wid * VT
    lane = lax.iota(jnp.int32, 16)

    def fire_in(c, in_ref, f_ref, sem):
        v0 = vbase + c * CH
        fa = (v0 // 8) * 8
        for s in range(F):
            pltpu.async_copy(emb_hbm.at[pl.ds((s * V + v0) * D, CH * D)],
                             in_ref.at[pl.ds(s * CH * D, CH * D)], sem)
        pltpu.async_copy(fc_hbm.at[pl.ds(fa, 32)], f_ref, sem)

    def wait_in(in_ref, f_ref, sem):
        for s in range(F):
            pltpu.make_async_copy(emb_hbm.at[pl.ds(s * V * D, CH * D)],
                                  in_ref.at[pl.ds(s * CH * D, CH * D)], sem).wait()
        pltpu.make_async_copy(fc_hbm.at[pl.ds(0, 32)], f_ref, sem).wait()

    def fire_out(c, o_ref, sem):
        pltpu.async_copy(o_ref,
                         tab_hbm.at[pl.ds((vbase + c * CH) * W, CH * W)], sem)

    def wait_out(o_ref, sem):
        pltpu.make_async_copy(o_ref,
                              tab_hbm.at[pl.ds(vbase * W, CH * W)], sem).wait()

    def interleave(c, in_ref, f_ref, o_ref):
        v0 = vbase + c * CH
        d0 = v0 - (v0 // 8) * 8
        def vbody(vv, carry):
            for s in range(F):
                o_ref[pl.ds(vv * W + s * D, D)] = \
                    in_ref[pl.ds((s * CH + vv) * D, D)]
            fi = d0 + vv
            foff = jnp.where(fi < 16, 0, 16)
            fpc = f_ref[pl.ds(foff, 16)]
            fv = _take16(fpc, jnp.full((16,), fi - foff, jnp.int32))
            o_ref[pl.ds(vv * W + F * D, D)] = fv
            return carry
        lax.fori_loop(0, CH, vbody, 0)

    fire_in(0, in0_v, f0_v, si0)
    fire_in(1, in1_v, f1_v, si1)

    def round_body(cc, carry):
        c0 = 2 * cc
        c1 = c0 + 1
        wait_in(in0_v, f0_v, si0)

        @pl.when(cc > 0)
        def _():
            wait_out(o0_v, so0)
        interleave(c0, in0_v, f0_v, o0_v)
        fire_out(c0, o0_v, so0)
        fire_in(jnp.minimum(c0 + 2, NCH - 1), in0_v, f0_v, si0)

        wait_in(in1_v, f1_v, si1)

        @pl.when(cc > 0)
        def _():
            wait_out(o1_v, so1)
        interleave(c1, in1_v, f1_v, o1_v)
        fire_out(c1, o1_v, so1)
        fire_in(jnp.minimum(c1 + 2, NCH - 1), in1_v, f1_v, si1)
        return carry

    lax.fori_loop(0, (NCH - 1) // 2, round_body, 0)

    wait_in(in0_v, f0_v, si0)
    wait_out(o0_v, so0)
    interleave(NCH - 1, in0_v, f0_v, o0_v)
    fire_out(NCH - 1, o0_v, so0)
    wait_in(in1_v, f1_v, si1)
    wait_out(o1_v, so1)
    wait_out(o0_v, so0)


def kernel(x, emb_tables, fc_table, bias):
    xflat = x.astype(jnp.int32).reshape(B * F)
    tab_aug = _relayout_sc(emb_tables.reshape(F * V * D),
                           fc_table.reshape(V)).reshape(V, W)
    bias16 = jnp.broadcast_to(bias.astype(jnp.float32), (16,))
    return _ffm_sc(xflat, tab_aug, bias16)


# speedup vs baseline: 2.7624x; 1.1339x over previous
import functools

import jax
import jax.numpy as jnp
from jax import lax
from jax.experimental import pallas as pl
from jax.experimental.pallas import tpu as pltpu
from jax.experimental.pallas import tpu_sc as plsc

F = 26
FP = 32
V = 100000
D = 16
B = 4096
NC = 2
NS = 16
NW = NC * NS
SPW = B // NW
W = F * D
G4 = 4
NG = SPW // G4


def _take16(vec, idx):
    return vec.at[idx].get(mode="promise_in_bounds")


@functools.partial(
    pl.kernel,
    out_type=jax.ShapeDtypeStruct((B,), jnp.float32),
    mesh=plsc.VectorSubcoreMesh(core_axis_name="c", subcore_axis_name="s"),
    compiler_params=pltpu.CompilerParams(use_tc_tiling_on_sc=False),
    scratch_types=[
        pltpu.VMEM((SPW * F,), jnp.int32),
        pltpu.VMEM((SPW * FP,), jnp.int32),
        pltpu.VMEM((G4 * F, W), jnp.float32),
        pltpu.VMEM((G4 * F, W), jnp.float32),
        pltpu.VMEM((SPW * FP,), jnp.float32),
        pltpu.VMEM((SPW,), jnp.float32),
        pltpu.VMEM((16,), jnp.float32),
        pltpu.SemaphoreType.DMA,
        pltpu.SemaphoreType.DMA,
        pltpu.SemaphoreType.DMA,
    ],
)
def _ffm_sc(x_hbm, x32_hbm, tab_hbm, fc_hbm, bias_hbm, out_hbm,
            x_v, x32_v, slabs0_v, slabs1_v, fc_v, out_v, bias_v,
            sem0, sem1, sem_fc):
    wid = lax.axis_index("s") * NC + lax.axis_index("c")
    base = wid * SPW

    pltpu.sync_copy(bias_hbm, bias_v)
    pltpu.sync_copy(x_hbm.at[pl.ds(base * F, SPW * F)], x_v)
    pltpu.sync_copy(x32_hbm.at[pl.ds(base * FP, SPW * FP)], x32_v)
    fc_copy = pltpu.async_copy(fc_hbm.at[x32_v], fc_v, sem_fc)

    lane = lax.iota(jnp.int32, 16)

    def fire(g, slabs_ref, sem):
        idx = x_v.at[pl.ds(g * (G4 * F), G4 * F)]
        pltpu.async_copy(tab_hbm.at[idx], slabs_ref, sem)

    def wait(g, slabs_ref, sem):
        idx = x_v.at[pl.ds(g * (G4 * F), G4 * F)]
        pltpu.make_async_copy(tab_hbm.at[idx], slabs_ref, sem).wait()

    def compute(g, slabs_ref, out_vec):
        for r in range(G4):
            b = g * G4 + r
            acc = jnp.zeros((16,), jnp.float32)
            for i in range(F - 1):
                for j in range(i + 1, F):
                    acc = acc + (slabs_ref[r * F + i, pl.ds(j * D, D)] *
                                 slabs_ref[r * F + j, pl.ds(i * D, D)])
            f0 = fc_v[pl.ds(b * FP, 16)]
            f1 = fc_v[pl.ds(b * FP + 16, 16)]
            acc = acc + f0 + jnp.where(lane < F - 16, f1, 0.0)
            for sh in (8, 4, 2, 1):
                acc = acc + _take16(acc, lane ^ sh)
            out_vec = jnp.where(lane == b % 16, acc, out_vec)
            out_v[pl.ds((b // 16) * 16, 16)] = out_vec
        return out_vec

    fire(0, slabs0_v, sem0)
    fc_copy.wait()

    def pair_body(gg, out_vec):
        g0 = 2 * gg
        g1 = g0 + 1
        fire(g1, slabs1_v, sem1)
        wait(g0, slabs0_v, sem0)
        out_vec = compute(g0, slabs0_v, out_vec)
        fire(jnp.minimum(g1 + 1, NG - 1), slabs0_v, sem0)
        wait(g1, slabs1_v, sem1)
        return compute(g1, slabs1_v, out_vec)

    lax.fori_loop(0, NG // 2, pair_body, jnp.zeros((16,), jnp.float32))
    wait(NG - 1, slabs0_v, sem0)

    bb = bias_v[:]
    for g in range(SPW // 16):
        zz = out_v[pl.ds(g * 16, 16)] + bb
        out_v[pl.ds(g * 16, 16)] = 1.0 / (1.0 + jnp.exp(-zz))
    pltpu.sync_copy(out_v, out_hbm.at[pl.ds(base, SPW)])


def kernel(x, emb_tables, fc_table, bias):
    x32 = x.astype(jnp.int32)
    xflat = x32.reshape(B * F)
    xpad = jnp.pad(x32, ((0, 0), (0, FP - F))).reshape(B * FP)
    tab_t = emb_tables.transpose(1, 0, 2).reshape(V, W)
    fc = fc_table.reshape(V)
    bias16 = jnp.broadcast_to(bias.astype(jnp.float32), (16,))
    return _ffm_sc(xflat, xpad, tab_t, fc, bias16)
